# baseline (device time: 56616 ns/iter reference)
import jax
import jax.numpy as jnp
from jax import lax
from jax.experimental import pallas as pl
from jax.experimental.pallas import tpu as pltpu

N_DEV = 4


def kernel(x, w_mat):
    m_per, k = x.shape
    _, n = w_mat.shape
    n_per = n // N_DEV

    def body(x_ref, w_hbm, out_ref, w_bufs, tile_bf, recv_bf,
             wdma_sems, send_sems, recv_sems):
        my_i = lax.axis_index("i")

        order = (1, 3, 2, 0)

        k_half = k // 2

        def w_fetch(step, slot):
            d = order[step]
            j = (my_i + d) % N_DEV
            cps = []
            for h in range(2):
                cp = pltpu.make_async_copy(
                    w_hbm.at[pl.ds(h * k_half, k_half),
                             pl.ds(j * n_per, n_per)],
                    w_bufs.at[slot, pl.ds(h * k_half, k_half)],
                    wdma_sems.at[slot, h],
                )
                cp.start()
                cps.append(cp)
            return cps

        fetches = [w_fetch(0, 0), w_fetch(1, 1), w_fetch(2, 2)]

        barrier_sem = pltpu.get_barrier_semaphore()
        for dev in range(N_DEV):
            @pl.when(my_i != dev)
            def _():
                pl.semaphore_signal(
                    barrier_sem, inc=1,
                    device_id=(dev,), device_id_type=pl.DeviceIdType.MESH,
                )
        pl.semaphore_wait(barrier_sem, N_DEV - 1)
        sends = []

        for step in range(N_DEV):
            slot = step % 3
            for cp in fetches[step]:
                cp.wait()
            t = jnp.dot(x_ref[:, :], w_bufs[slot],
                        preferred_element_type=jnp.float32)
            t = t * jax.nn.sigmoid(t)
            d = order[step]
            if d == 0:
                out_ref[pl.ds(my_i * m_per, m_per), :] = t
            else:
                tile_bf[d - 1, :, :] = t.astype(jnp.bfloat16)
                rdma = pltpu.make_async_remote_copy(
                    src_ref=tile_bf.at[d - 1],
                    dst_ref=recv_bf.at[d - 1],
                    send_sem=send_sems.at[d - 1],
                    recv_sem=recv_sems.at[d - 1],
                    device_id=((my_i + d) % N_DEV,),
                    device_id_type=pl.DeviceIdType.MESH,
                )
                rdma.start()
                sends.append(rdma)
            if step + 3 < N_DEV:
                fetches.append(w_fetch(step + 3, slot))

        for d in (1, 3, 2):
            src = (my_i - d) % N_DEV
            recv = pltpu.make_async_remote_copy(
                src_ref=tile_bf.at[d - 1],
                dst_ref=recv_bf.at[d - 1],
                send_sem=send_sems.at[d - 1],
                recv_sem=recv_sems.at[d - 1],
                device_id=(src,),
                device_id_type=pl.DeviceIdType.MESH,
            )
            recv.wait_recv()
            out_ref[pl.ds(src * m_per, m_per), :] = (
                recv_bf[d - 1].astype(jnp.float32)
            )
        for rdma in sends:
            rdma.wait_send()

    return pl.pallas_call(
        body,
        out_shape=jax.ShapeDtypeStruct((N_DEV * m_per, n_per), jnp.float32),
        in_specs=[
            pl.BlockSpec(memory_space=pltpu.VMEM),
            pl.BlockSpec(memory_space=pltpu.MemorySpace.HBM),
        ],
        out_specs=pl.BlockSpec(memory_space=pltpu.VMEM),
        scratch_shapes=[
            pltpu.VMEM((3, k, n_per), jnp.float32),
            pltpu.VMEM((N_DEV - 1, m_per, n_per), jnp.bfloat16),
            pltpu.VMEM((N_DEV - 1, m_per, n_per), jnp.bfloat16),
            pltpu.SemaphoreType.DMA((3, 2)),
            pltpu.SemaphoreType.DMA((N_DEV - 1,)),
            pltpu.SemaphoreType.DMA((N_DEV - 1,)),
        ],
        compiler_params=pltpu.CompilerParams(
            collective_id=0,
            vmem_limit_bytes=128 * 1024 * 1024,
        ),
    )(x, w_mat)


# device time: 49268 ns/iter; 1.1491x vs baseline; 1.1491x over previous
import jax
import jax.numpy as jnp
from jax import lax
from jax.experimental import pallas as pl
from jax.experimental.pallas import tpu as pltpu

N_DEV = 4


def kernel(x, w_mat):
    m_per, k = x.shape
    _, n = w_mat.shape
    n_per = n // N_DEV

    def body(x_ref, w_hbm, out_ref, w_bufs, tile_bf, recv_bf,
             wdma_sems, send_sems, recv_sems):
        my_i = lax.axis_index("i")

        order = (2, 1, 3, 0)
        m_half = m_per // 2

        k_half = k // 2

        def w_fetch(step, slot):
            d = order[step]
            j = (my_i + d) % N_DEV
            cps = []
            for h in range(2):
                cp = pltpu.make_async_copy(
                    w_hbm.at[pl.ds(h * k_half, k_half),
                             pl.ds(j * n_per, n_per)],
                    w_bufs.at[slot, pl.ds(h * k_half, k_half)],
                    wdma_sems.at[slot, h],
                )
                cp.start()
                cps.append(cp)
            return cps

        fetches = [w_fetch(0, 0), w_fetch(1, 1), w_fetch(2, 2)]

        barrier_sem = pltpu.get_barrier_semaphore()
        for dev in range(N_DEV):
            @pl.when(my_i != dev)
            def _():
                pl.semaphore_signal(
                    barrier_sem, inc=1,
                    device_id=(dev,), device_id_type=pl.DeviceIdType.MESH,
                )
        pl.semaphore_wait(barrier_sem, N_DEV - 1)
        sends = []

        for step in range(N_DEV):
            slot = step % 3
            for cp in fetches[step]:
                cp.wait()
            d = order[step]
            for h in range(2):
                t = jnp.dot(x_ref[pl.ds(h * m_half, m_half), :],
                            w_bufs[slot],
                            preferred_element_type=jnp.float32)
                t = t * jax.nn.sigmoid(t)
                if d == 0:
                    out_ref[pl.ds(my_i * m_per + h * m_half, m_half), :] = t
                else:
                    tile_bf[d - 1, pl.ds(h * m_half, m_half), :] = (
                        t.astype(jnp.bfloat16)
                    )
                    rdma = pltpu.make_async_remote_copy(
                        src_ref=tile_bf.at[d - 1, pl.ds(h * m_half, m_half)],
                        dst_ref=recv_bf.at[d - 1, pl.ds(h * m_half, m_half)],
                        send_sem=send_sems.at[d - 1, h],
                        recv_sem=recv_sems.at[d - 1, h],
                        device_id=((my_i + d) % N_DEV,),
                        device_id_type=pl.DeviceIdType.MESH,
                    )
                    rdma.start()
                    sends.append(rdma)
            if step + 3 < N_DEV:
                fetches.append(w_fetch(step + 3, slot))

        for d in (2, 1, 3):
            src = (my_i - d) % N_DEV
            for h in range(2):
                recv = pltpu.make_async_remote_copy(
                    src_ref=tile_bf.at[d - 1, pl.ds(h * m_half, m_half)],
                    dst_ref=recv_bf.at[d - 1, pl.ds(h * m_half, m_half)],
                    send_sem=send_sems.at[d - 1, h],
                    recv_sem=recv_sems.at[d - 1, h],
                    device_id=(src,),
                    device_id_type=pl.DeviceIdType.MESH,
                )
                recv.wait_recv()
                out_ref[pl.ds(src * m_per + h * m_half, m_half), :] = (
                    recv_bf[d - 1, pl.ds(h * m_half, m_half), :]
                    .astype(jnp.float32)
                )
        for rdma in sends:
            rdma.wait_send()

    return pl.pallas_call(
        body,
        out_shape=jax.ShapeDtypeStruct((N_DEV * m_per, n_per), jnp.float32),
        in_specs=[
            pl.BlockSpec(memory_space=pltpu.VMEM),
            pl.BlockSpec(memory_space=pltpu.MemorySpace.HBM),
        ],
        out_specs=pl.BlockSpec(memory_space=pltpu.VMEM),
        scratch_shapes=[
            pltpu.VMEM((3, k, n_per), jnp.float32),
            pltpu.VMEM((N_DEV - 1, m_per, n_per), jnp.bfloat16),
            pltpu.VMEM((N_DEV - 1, m_per, n_per), jnp.bfloat16),
            pltpu.SemaphoreType.DMA((3, 2)),
            pltpu.SemaphoreType.DMA((N_DEV - 1, 2)),
            pltpu.SemaphoreType.DMA((N_DEV - 1, 2)),
        ],
        compiler_params=pltpu.CompilerParams(
            collective_id=0,
            vmem_limit_bytes=128 * 1024 * 1024,
        ),
    )(x, w_mat)


# device time: 48014 ns/iter; 1.1792x vs baseline; 1.0261x over previous
import jax
import jax.numpy as jnp
from jax import lax
from jax.experimental import pallas as pl
from jax.experimental.pallas import tpu as pltpu

N_DEV = 4


def kernel(x, w_mat):
    m_per, k = x.shape
    _, n = w_mat.shape
    n_per = n // N_DEV

    def body(x_ref, w_hbm, out_ref, w_bufs, tile_bf, recv_bf,
             wdma_sems, send_sems, recv_sems):
        my_i = lax.axis_index("i")

        order = (2, 1, 3, 0)
        m_half = m_per // 4

        k_half = k // 2

        def w_fetch(step, slot):
            d = order[step]
            j = (my_i + d) % N_DEV
            cps = []
            for h in range(2):
                cp = pltpu.make_async_copy(
                    w_hbm.at[pl.ds(h * k_half, k_half),
                             pl.ds(j * n_per, n_per)],
                    w_bufs.at[slot, pl.ds(h * k_half, k_half)],
                    wdma_sems.at[slot, h],
                )
                cp.start()
                cps.append(cp)
            return cps

        fetches = [w_fetch(0, 0), w_fetch(1, 1), w_fetch(2, 2)]

        barrier_sem = pltpu.get_barrier_semaphore()
        for dev in range(N_DEV):
            @pl.when(my_i != dev)
            def _():
                pl.semaphore_signal(
                    barrier_sem, inc=1,
                    device_id=(dev,), device_id_type=pl.DeviceIdType.MESH,
                )
        pl.semaphore_wait(barrier_sem, N_DEV - 1)
        sends = []

        for step in range(N_DEV):
            slot = step % 3
            for cp in fetches[step]:
                cp.wait()
            d = order[step]
            for h in range(4):
                t = jnp.dot(x_ref[pl.ds(h * m_half, m_half), :],
                            w_bufs[slot],
                            preferred_element_type=jnp.float32)
                t = t * jax.nn.sigmoid(t)
                if d == 0:
                    out_ref[pl.ds(my_i * m_per + h * m_half, m_half), :] = t
                else:
                    tile_bf[d - 1, pl.ds(h * m_half, m_half), :] = (
                        t.astype(jnp.bfloat16)
                    )
                    rdma = pltpu.make_async_remote_copy(
                        src_ref=tile_bf.at[d - 1, pl.ds(h * m_half, m_half)],
                        dst_ref=recv_bf.at[d - 1, pl.ds(h * m_half, m_half)],
                        send_sem=send_sems.at[d - 1, h],
                        recv_sem=recv_sems.at[d - 1, h],
                        device_id=((my_i + d) % N_DEV,),
                        device_id_type=pl.DeviceIdType.MESH,
                    )
                    rdma.start()
                    sends.append(rdma)
            if step + 3 < N_DEV:
                fetches.append(w_fetch(step + 3, slot))

        for d in (2, 1, 3):
            src = (my_i - d) % N_DEV
            for h in range(4):
                recv = pltpu.make_async_remote_copy(
                    src_ref=tile_bf.at[d - 1, pl.ds(h * m_half, m_half)],
                    dst_ref=recv_bf.at[d - 1, pl.ds(h * m_half, m_half)],
                    send_sem=send_sems.at[d - 1, h],
                    recv_sem=recv_sems.at[d - 1, h],
                    device_id=(src,),
                    device_id_type=pl.DeviceIdType.MESH,
                )
                recv.wait_recv()
                out_ref[pl.ds(src * m_per + h * m_half, m_half), :] = (
                    recv_bf[d - 1, pl.ds(h * m_half, m_half), :]
                    .astype(jnp.float32)
                )
        for rdma in sends:
            rdma.wait_send()

    return pl.pallas_call(
        body,
        out_shape=jax.ShapeDtypeStruct((N_DEV * m_per, n_per), jnp.float32),
        in_specs=[
            pl.BlockSpec(memory_space=pltpu.VMEM),
            pl.BlockSpec(memory_space=pltpu.MemorySpace.HBM),
        ],
        out_specs=pl.BlockSpec(memory_space=pltpu.VMEM),
        scratch_shapes=[
            pltpu.VMEM((3, k, n_per), jnp.float32),
            pltpu.VMEM((N_DEV - 1, m_per, n_per), jnp.bfloat16),
            pltpu.VMEM((N_DEV - 1, m_per, n_per), jnp.bfloat16),
            pltpu.SemaphoreType.DMA((3, 2)),
            pltpu.SemaphoreType.DMA((N_DEV - 1, 4)),
            pltpu.SemaphoreType.DMA((N_DEV - 1, 4)),
        ],
        compiler_params=pltpu.CompilerParams(
            collective_id=0,
            vmem_limit_bytes=128 * 1024 * 1024,
        ),
    )(x, w_mat)
